# baseline (speedup 1.0000x reference)
import functools

import jax
import jax.numpy as jnp
from jax import lax
from jax.experimental import pallas as pl
from jax.experimental.pallas import tpu as pltpu
from jax.experimental.pallas import tpu_sc as plsc

_N_USERS = 1000000
_EMBED = 16
_B = 16384
_ROWS = 2000000
_TI = _ROWS // 128
_HALF = _TI * 8 * 128
_OHALF = (_B // 128) * 8 * 128

_info = plsc.get_sparse_core_info()
_NC = _info.num_cores
_NS = _info.num_subcores
_L = _info.num_lanes
_NW = _NC * _NS
_BPW = _B // _NW
_WPW = _BPW * 8

_mesh = plsc.VectorSubcoreMesh(core_axis_name="c", subcore_axis_name="s")


@functools.partial(
    pl.kernel,
    mesh=_mesh,
    compiler_params=pltpu.CompilerParams(use_tc_tiling_on_sc=False),
    out_type=[
        jax.ShapeDtypeStruct((2 * _OHALF,), jnp.float32),
        jax.ShapeDtypeStruct((2 * _OHALF,), jnp.float32),
        jax.ShapeDtypeStruct((2 * _OHALF,), jnp.float32),
    ],
    scratch_types=[
        pltpu.VMEM((_BPW,), jnp.int32),
        pltpu.VMEM((_BPW,), jnp.int32),
        pltpu.VMEM((_BPW,), jnp.int32),
        pltpu.VMEM((2, _WPW), jnp.int32),
        pltpu.VMEM((2, _WPW), jnp.int32),
        pltpu.VMEM((2, _WPW), jnp.int32),
        pltpu.VMEM((2, _WPW), jnp.float32),
        pltpu.VMEM((2, _WPW), jnp.float32),
        pltpu.VMEM((2, _WPW), jnp.float32),
        pltpu.SemaphoreType.DMA,
        pltpu.SemaphoreType.DMA,
    ],
)
def _mf_gather(users_hbm, pos_hbm, neg_hbm, table_hbm,
               u_out, p_out, n_out,
               stage_u, stage_p, stage_n,
               addr_u, addr_p, addr_n, rows_u, rows_p, rows_n,
               sem, sem2):
    wid = lax.axis_index("s") * _NC + lax.axis_index("c")
    base = wid * _BPW

    stagings = [
        pltpu.async_copy(src.at[pl.ds(base, _BPW)], dst, sem2)
        for src, dst in ((users_hbm, stage_u), (pos_hbm, stage_p),
                         (neg_hbm, stage_n))
    ]

    def compute_and_fire(staging, stage, addrs, rows, item_offset):
        staging.wait()

        def body(m, _):
            v = stage[pl.ds(m * _L, _L)] + item_offset
            b = ((v >> 7) << 10) + (v & 127)
            c = (m >> 3) << 10
            lv = (m & 7) << 4
            for f in range(8):
                addrs[0, pl.ds(c + f * 128 + lv, _L)] = b + f * 128
                addrs[1, pl.ds(c + f * 128 + lv, _L)] = b + (_HALF + f * 128)
            return 0

        copies = []
        for k in range(4):
            lax.fori_loop(k * 8, (k + 1) * 8, body, 0)
            w = pl.ds(k * 1024, 1024)
            copies.append([
                pltpu.async_copy(
                    table_hbm.at[addrs.at[tf, w]], rows.at[tf, w], sem)
                for tf in range(2)
            ])
        return copies

    cu = compute_and_fire(stagings[0], stage_u, addr_u, rows_u, 0)
    cp = compute_and_fire(stagings[1], stage_p, addr_p, rows_p, _N_USERS)
    cn = compute_and_fire(stagings[2], stage_n, addr_n, rows_n, _N_USERS)

    outs = []
    for copies, rows, out in ((cu, rows_u, u_out), (cp, rows_p, p_out),
                              (cn, rows_n, n_out)):
        for k in range(4):
            for tf in range(2):
                copies[k][tf].wait()
        for tf in range(2):
            outs.append(pltpu.async_copy(
                rows.at[tf],
                out.at[pl.ds(tf * _OHALF + wid * _WPW, _WPW)],
                sem2,
            ))
    for c in outs:
        c.wait()


def kernel(users, pos_items, neg_items, weight):
    wphys = (weight.reshape(_TI, 128, 2, 8)
                   .transpose(2, 0, 3, 1)
                   .reshape(2 * _HALF))
    u_f, p_f, n_f = _mf_gather(
        users.astype(jnp.int32),
        pos_items.astype(jnp.int32),
        neg_items.astype(jnp.int32),
        wphys,
    )

    def unflatten(o):
        return (o.reshape(2, _B // 128, 8, 128)
                 .transpose(1, 3, 0, 2)
                 .reshape(_B, _EMBED))

    return (unflatten(u_f), unflatten(p_f), unflatten(n_f))


# speedup vs baseline: 1.0147x; 1.0147x over previous
import functools

import jax
import jax.numpy as jnp
from jax import lax
from jax.experimental import pallas as pl
from jax.experimental.pallas import tpu as pltpu
from jax.experimental.pallas import tpu_sc as plsc

_N_USERS = 1000000
_EMBED = 16
_B = 16384
_ROWS = 2000000
_TI = _ROWS // 128
_HALF = _TI * 8 * 128
_OHALF = (_B // 128) * 8 * 128

_info = plsc.get_sparse_core_info()
_NC = _info.num_cores
_NS = _info.num_subcores
_L = _info.num_lanes
_NW = _NC * _NS
_BPW = _B // _NW
_WPW = _BPW * 8

_mesh = plsc.VectorSubcoreMesh(core_axis_name="c", subcore_axis_name="s")


@functools.partial(
    pl.kernel,
    mesh=_mesh,
    compiler_params=pltpu.CompilerParams(use_tc_tiling_on_sc=False),
    out_type=[
        jax.ShapeDtypeStruct((2 * _OHALF,), jnp.float32),
        jax.ShapeDtypeStruct((2 * _OHALF,), jnp.float32),
        jax.ShapeDtypeStruct((2 * _OHALF,), jnp.float32),
    ],
    scratch_types=[
        pltpu.VMEM((_BPW,), jnp.int32),
        pltpu.VMEM((_BPW,), jnp.int32),
        pltpu.VMEM((_BPW,), jnp.int32),
        pltpu.VMEM((2, _WPW), jnp.int32),
        pltpu.VMEM((2, _WPW), jnp.int32),
        pltpu.VMEM((2, _WPW), jnp.int32),
        pltpu.VMEM((2, _WPW), jnp.float32),
        pltpu.VMEM((2, _WPW), jnp.float32),
        pltpu.VMEM((2, _WPW), jnp.float32),
        pltpu.SemaphoreType.DMA,
        pltpu.SemaphoreType.DMA,
    ],
)
def _mf_gather(users_hbm, pos_hbm, neg_hbm, table_hbm,
               u_out, p_out, n_out,
               stage_u, stage_p, stage_n,
               addr_u, addr_p, addr_n, rows_u, rows_p, rows_n,
               sem, sem2):
    wid = lax.axis_index("s") * _NC + lax.axis_index("c")
    base = wid * _BPW

    stagings = [
        pltpu.async_copy(src.at[pl.ds(base, _BPW)], dst, sem2)
        for src, dst in ((users_hbm, stage_u), (pos_hbm, stage_p),
                         (neg_hbm, stage_n))
    ]

    def compute_addrs(staging, stage, addrs, item_offset):
        staging.wait()

        def body(m, _):
            v = stage[pl.ds(m * _L, _L)] + item_offset
            b = ((v >> 7) << 10) + (v & 127)
            c = (m >> 3) << 10
            lv = (m & 7) << 4
            for f in range(8):
                addrs[0, pl.ds(c + f * 128 + lv, _L)] = b + f * 128
                addrs[1, pl.ds(c + f * 128 + lv, _L)] = b + (_HALF + f * 128)
            return 0
        lax.fori_loop(0, _BPW // _L, body, 0)

    def fire(addrs, rows):
        return [
            pltpu.async_copy(table_hbm.at[addrs.at[tf]], rows.at[tf], sem)
            for tf in range(2)
        ]

    compute_addrs(stagings[0], stage_u, addr_u, 0)
    cu = fire(addr_u, rows_u)
    compute_addrs(stagings[1], stage_p, addr_p, _N_USERS)
    cp = fire(addr_p, rows_p)
    compute_addrs(stagings[2], stage_n, addr_n, _N_USERS)
    cn = fire(addr_n, rows_n)

    outs = []
    for copies, rows, out in ((cu, rows_u, u_out), (cp, rows_p, p_out),
                              (cn, rows_n, n_out)):
        for tf in range(2):
            copies[tf].wait()
            outs.append(pltpu.async_copy(
                rows.at[tf],
                out.at[pl.ds(tf * _OHALF + wid * _WPW, _WPW)],
                sem2,
            ))
    for c in outs:
        c.wait()


def kernel(users, pos_items, neg_items, weight):
    wphys = (weight.reshape(_TI, 128, 2, 8)
                   .transpose(2, 0, 3, 1)
                   .reshape(2 * _HALF))
    u_f, p_f, n_f = _mf_gather(
        users.astype(jnp.int32),
        pos_items.astype(jnp.int32),
        neg_items.astype(jnp.int32),
        wphys,
    )

    def unflatten(o):
        return (o.reshape(2, _B // 128, 8, 128)
                 .transpose(1, 3, 0, 2)
                 .reshape(_B, _EMBED))

    return (unflatten(u_f), unflatten(p_f), unflatten(n_f))


# speedup vs baseline: 1.0151x; 1.0004x over previous
import functools

import jax
import jax.numpy as jnp
from jax import lax
from jax.experimental import pallas as pl
from jax.experimental.pallas import tpu as pltpu
from jax.experimental.pallas import tpu_sc as plsc

_N_USERS = 1000000
_EMBED = 16
_B = 16384
_ROWS = 2000000
_TI = _ROWS // 128
_HALF = _TI * 8 * 128
_OHALF = (_B // 128) * 8 * 128

_info = plsc.get_sparse_core_info()
_NC = _info.num_cores
_NS = _info.num_subcores
_L = _info.num_lanes
_NW = _NC * _NS
_BPW = _B // _NW
_WPW = _BPW * 8

_mesh = plsc.VectorSubcoreMesh(core_axis_name="c", subcore_axis_name="s")


@functools.partial(
    pl.kernel,
    mesh=_mesh,
    compiler_params=pltpu.CompilerParams(
        use_tc_tiling_on_sc=False,
        disable_bounds_checks=True,
        skip_device_barrier=True,
    ),
    out_type=[
        jax.ShapeDtypeStruct((2 * _OHALF,), jnp.float32),
        jax.ShapeDtypeStruct((2 * _OHALF,), jnp.float32),
        jax.ShapeDtypeStruct((2 * _OHALF,), jnp.float32),
    ],
    scratch_types=[
        pltpu.VMEM((_BPW,), jnp.int32),
        pltpu.VMEM((_BPW,), jnp.int32),
        pltpu.VMEM((_BPW,), jnp.int32),
        pltpu.VMEM((2, _WPW), jnp.int32),
        pltpu.VMEM((2, _WPW), jnp.int32),
        pltpu.VMEM((2, _WPW), jnp.int32),
        pltpu.VMEM((2, _WPW), jnp.float32),
        pltpu.VMEM((2, _WPW), jnp.float32),
        pltpu.VMEM((2, _WPW), jnp.float32),
        pltpu.SemaphoreType.DMA,
        pltpu.SemaphoreType.DMA,
    ],
)
def _mf_gather(users_hbm, pos_hbm, neg_hbm, table_hbm,
               u_out, p_out, n_out,
               stage_u, stage_p, stage_n,
               addr_u, addr_p, addr_n, rows_u, rows_p, rows_n,
               sem, sem2):
    wid = lax.axis_index("s") * _NC + lax.axis_index("c")
    base = wid * _BPW

    stagings = [
        pltpu.async_copy(src.at[pl.ds(base, _BPW)], dst, sem2)
        for src, dst in ((users_hbm, stage_u), (pos_hbm, stage_p),
                         (neg_hbm, stage_n))
    ]

    def compute_addrs(staging, stage, addrs, item_offset):
        staging.wait()

        def body(m, _):
            v = stage[pl.ds(m * _L, _L)] + item_offset
            b = ((v >> 7) << 10) + (v & 127)
            c = (m >> 3) << 10
            lv = (m & 7) << 4
            for f in range(8):
                addrs[0, pl.ds(c + f * 128 + lv, _L)] = b + f * 128
                addrs[1, pl.ds(c + f * 128 + lv, _L)] = b + (_HALF + f * 128)
            return 0
        lax.fori_loop(0, _BPW // _L, body, 0)

    def fire(addrs, rows):
        return [
            pltpu.async_copy(table_hbm.at[addrs.at[tf]], rows.at[tf], sem)
            for tf in range(2)
        ]

    compute_addrs(stagings[0], stage_u, addr_u, 0)
    cu = fire(addr_u, rows_u)
    compute_addrs(stagings[1], stage_p, addr_p, _N_USERS)
    cp = fire(addr_p, rows_p)
    compute_addrs(stagings[2], stage_n, addr_n, _N_USERS)
    cn = fire(addr_n, rows_n)

    outs = []
    for copies, rows, out in ((cu, rows_u, u_out), (cp, rows_p, p_out),
                              (cn, rows_n, n_out)):
        for tf in range(2):
            copies[tf].wait()
            outs.append(pltpu.async_copy(
                rows.at[tf],
                out.at[pl.ds(tf * _OHALF + wid * _WPW, _WPW)],
                sem2,
            ))
    for c in outs:
        c.wait()


def kernel(users, pos_items, neg_items, weight):
    wphys = (weight.reshape(_TI, 128, 2, 8)
                   .transpose(2, 0, 3, 1)
                   .reshape(2 * _HALF))
    u_f, p_f, n_f = _mf_gather(
        users.astype(jnp.int32),
        pos_items.astype(jnp.int32),
        neg_items.astype(jnp.int32),
        wphys,
    )

    def unflatten(o):
        return (o.reshape(2, _B // 128, 8, 128)
                 .transpose(1, 3, 0, 2)
                 .reshape(_B, _EMBED))

    return (unflatten(u_f), unflatten(p_f), unflatten(n_f))
